# baseline (speedup 1.0000x reference)
import jax
import jax.numpy as jnp
from jax import lax
from jax.experimental import pallas as pl
from jax.experimental.pallas import tpu as pltpu
from jax.experimental.pallas import tpu_sc as plsc

_NC = 2
_NS = 16
_NW = _NC * _NS

_BATCH = 4096
_HIST = 200
_V = 1000000
_D = 64

_IW = 128
_NUNIT = _BATCH * _HIST // _IW
_UPW = _NUNIT // _NW
_NB2 = _BATCH // _IW
_GU = 2
_GPW = _UPW // _GU


def _body(wp_hbm, idx_hbm, out_hbm, idx_v, pidx_v, p_v, t_v, sem_g, sem_o):
    wid = lax.axis_index("s") * _NC + lax.axis_index("c")
    u0 = wid * _UPW
    iota = lax.iota(jnp.int32, 16)

    pltpu.sync_copy(idx_hbm.at[pl.ds(u0, _UPW)], idx_v)

    def prep_and_fire(s, buf):
        for j in range(_GU):
            ul = s * _GU + j
            for g in range(8):
                idxr = idx_v[ul, pl.ds(g * 16, 16)]
                pidx_v[buf * _GU + j, pl.ds(g * 16, 16)] = (
                    lax.shift_right_logical(idxr, 1)
                )
            pltpu.async_copy(
                wp_hbm.at[pidx_v.at[buf * _GU + j]],
                p_v.at[pl.ds((buf * _GU + j) * _IW, _IW)],
                sem_g,
            )

    def drain_gathers(buf):
        for j in range(_GU):
            pltpu.make_async_copy(
                wp_hbm.at[pidx_v.at[buf * _GU + j]],
                p_v.at[pl.ds((buf * _GU + j) * _IW, _IW)],
                sem_g,
            ).wait()

    def transpose_group(s, buf):
        for j in range(_GU):
            ul = s * _GU + j
            rowvs = []
            colvs = []
            for g in range(8):
                rowvs.append(iota + ((buf * _GU + j) * _IW + g * 16))
                idxr = idx_v[ul, pl.ds(g * 16, 16)]
                colvs.append(lax.shift_left(lax.bitwise_and(idxr, 1), 6))

            @plsc.parallel_loop(0, _D, unroll=16)
            def _d_loop(d):
                d2 = lax.shift_right_logical(d, 3)
                d1 = lax.bitwise_and(d, 7)
                for g in range(8):
                    v = plsc.load_gather(p_v, [rowvs[g], colvs[g] + d])
                    t_v[buf, d2, j, d1, pl.ds(g * 16, 16)] = v

    def fire_store(s, buf):
        u = u0 + s * _GU
        h = u // _NB2
        b2 = lax.rem(u, _NB2)
        pltpu.async_copy(
            t_v.at[buf], out_hbm.at[h, :, pl.ds(b2, _GU)], sem_o
        )

    def wait_store(s, buf):
        u = u0 + s * _GU
        h = u // _NB2
        b2 = lax.rem(u, _NB2)
        pltpu.make_async_copy(
            t_v.at[buf], out_hbm.at[h, :, pl.ds(b2, _GU)], sem_o
        ).wait()


    def pair(ss, carry):
        for b in range(2):
            s = ss * 2 + b
            nb = 1 - b


            @pl.when(s >= 2)
            def _free_tbuf():
                wait_store(s - 2, b)

            transpose_group(s, b)
            fire_store(s, b)
        return carry

    lax.fori_loop(0, _GPW // 2, pair, 0)
    wait_store(_GPW - 2, 0)
    wait_store(_GPW - 1, 1)


@jax.jit
def _embedding_lookup(input_, weight):
    wp = weight.reshape(_V // 2, 2 * _D)
    idx2 = input_.astype(jnp.int32).T.reshape(_NUNIT, _IW)
    mesh = plsc.VectorSubcoreMesh(core_axis_name="c", subcore_axis_name="s")
    out5 = pl.kernel(
        _body,
        out_type=jax.ShapeDtypeStruct((_HIST, 8, _NB2, 8, _IW), jnp.float32),
        mesh=mesh,
        scratch_types=[
            pltpu.VMEM((_UPW, _IW), jnp.int32),
            pltpu.VMEM((2 * _GU, _IW), jnp.int32),
            pltpu.VMEM((2 * _GU * _IW, 2 * _D), jnp.float32),
            pltpu.VMEM((2, 8, _GU, 8, _IW), jnp.float32),
            pltpu.SemaphoreType.DMA,
            pltpu.SemaphoreType.DMA,
        ],
        compiler_params=pltpu.CompilerParams(
            use_tc_tiling_on_sc=True, needs_layout_passes=False
        ),
    )(wp, idx2)
    return out5.transpose(2, 4, 0, 1, 3).reshape(_BATCH, _HIST, _D)


def kernel(input_, weight):
    return _embedding_lookup(input_, weight)


# speedup vs baseline: 1.0057x; 1.0057x over previous
import jax
import jax.numpy as jnp
from jax import lax
from jax.experimental import pallas as pl
from jax.experimental.pallas import tpu as pltpu
from jax.experimental.pallas import tpu_sc as plsc

_NC = 2
_NS = 16
_NW = _NC * _NS

_BATCH = 4096
_HIST = 200
_V = 1000000
_D = 64

_IW = 128
_NUNIT = _BATCH * _HIST // _IW
_UPW = _NUNIT // _NW
_NB2 = _BATCH // _IW
_TP = 129


def _body(wp_hbm, idx_hbm, out_hbm, idx_v, p_v, t_v, sem_g, sem_o):
    wid = lax.axis_index("s") * _NC + lax.axis_index("c")
    u0 = wid * _UPW
    iota = lax.iota(jnp.int32, 16)

    pltpu.sync_copy(idx_hbm.at[pl.ds(u0, _UPW)], idx_v)

    def fire_gather(ul, buf):
        pltpu.async_copy(
            wp_hbm.at[idx_v.at[ul]],
            p_v.at[pl.ds(buf * _IW, _IW)],
            sem_g,
        )

    def drain_gather(ul, buf):
        pltpu.make_async_copy(
            wp_hbm.at[idx_v.at[ul]],
            p_v.at[pl.ds(buf * _IW, _IW)],
            sem_g,
        ).wait()

    d2vs = [lax.shift_right_logical(iota + m * 16, 3) for m in range(4)]
    d1vs = [lax.bitwise_and(iota + m * 16, 7) for m in range(4)]
    zero16 = lax.bitwise_and(iota, 0)

    def transpose_unit(buf):
        bufv = zero16 + buf
        rowbase = buf * _IW

        @plsc.parallel_loop(0, _IW, unroll=8)
        def _k_loop(k):
            kv = zero16 + k
            for m in range(4):
                v = p_v[rowbase + k, pl.ds(m * 16, 16)]
                plsc.store_scatter(t_v, [bufv, d2vs[m], d1vs[m], kv], v)

    def fire_store(ul, buf):
        u = u0 + ul
        h = u // _NB2
        b2 = lax.rem(u, _NB2)
        pltpu.async_copy(
            t_v.at[buf, :, :, pl.ds(0, _IW)],
            out_hbm.at[h, :, b2],
            sem_o,
        )

    def wait_store(ul, buf):
        u = u0 + ul
        h = u // _NB2
        b2 = lax.rem(u, _NB2)
        pltpu.make_async_copy(
            t_v.at[buf, :, :, pl.ds(0, _IW)],
            out_hbm.at[h, :, b2],
            sem_o,
        ).wait()

    fire_gather(0, 0)

    def pair(uu, carry):
        for b in range(2):
            ul = uu * 2 + b
            nb = 1 - b

            @pl.when(ul < _UPW - 1)
            def _fire_next():
                fire_gather(ul + 1, nb)

            drain_gather(ul, b)

            @pl.when(ul >= 2)
            def _free_tbuf():
                wait_store(ul - 2, b)

            transpose_unit(b)
            fire_store(ul, b)
        return carry

    lax.fori_loop(0, _UPW // 2, pair, 0)
    wait_store(_UPW - 2, 0)
    wait_store(_UPW - 1, 1)


@jax.jit
def _embedding_lookup(input_, weight):
    wp = jnp.pad(weight, ((0, 0), (0, _D)))
    idx2 = input_.astype(jnp.int32).T.reshape(_NUNIT, _IW)
    mesh = plsc.VectorSubcoreMesh(core_axis_name="c", subcore_axis_name="s")
    out5 = pl.kernel(
        _body,
        out_type=jax.ShapeDtypeStruct((_HIST, 8, _NB2, 8, _IW), jnp.float32),
        mesh=mesh,
        scratch_types=[
            pltpu.VMEM((_UPW, _IW), jnp.int32),
            pltpu.VMEM((2 * _IW, 2 * _D), jnp.float32),
            pltpu.VMEM((2, 8, 8, _TP), jnp.float32),
            pltpu.SemaphoreType.DMA,
            pltpu.SemaphoreType.DMA,
        ],
        compiler_params=pltpu.CompilerParams(
            use_tc_tiling_on_sc=True, needs_layout_passes=False
        ),
    )(wp, idx2)
    return out5.transpose(2, 4, 0, 1, 3).reshape(_BATCH, _HIST, _D)


def kernel(input_, weight):
    return _embedding_lookup(input_, weight)


# speedup vs baseline: 1.2390x; 1.2320x over previous
import jax
import jax.numpy as jnp
from jax import lax
from jax.experimental import pallas as pl
from jax.experimental.pallas import tpu as pltpu
from jax.experimental.pallas import tpu_sc as plsc

_NC = 2
_NS = 16
_NW = _NC * _NS

_BATCH = 4096
_HIST = 200
_V = 1000000
_D = 64
_W = 2 * _D

_B = _BATCH * _HIST
_IW = 128
_BPW = _B // _NW
_KALL = _BPW // _IW
_C = 256
_K = _C // _IW
_NCHUNK = _BPW // _C


def _body(wp_hbm, idx_hbm, out_hbm, idx_v, p_v, sem_g, sem_o):
    wid = lax.axis_index("s") * _NC + lax.axis_index("c")
    row0 = wid * _KALL
    base0 = wid * _BPW

    pltpu.sync_copy(idx_hbm.at[pl.ds(row0, _KALL)], idx_v)

    def fire_gathers(g, buf):
        for j in range(_K):
            pltpu.async_copy(
                wp_hbm.at[idx_v.at[g * _K + j]],
                p_v.at[pl.ds(buf * _C + j * _IW, _IW)],
                sem_g,
            )

    def drain_gathers(g, buf):
        for j in range(_K):
            pltpu.make_async_copy(
                wp_hbm.at[idx_v.at[g * _K + j]],
                p_v.at[pl.ds(buf * _C + j * _IW, _IW)],
                sem_g,
            ).wait()

    def fire_store(g, buf):
        pltpu.async_copy(
            p_v.at[pl.ds(buf * _C, _C)],
            out_hbm.at[pl.ds(base0 + g * _C, _C)],
            sem_o,
        )

    def wait_store(g, buf):
        pltpu.make_async_copy(
            p_v.at[pl.ds(buf * _C, _C)],
            out_hbm.at[pl.ds(base0 + g * _C, _C)],
            sem_o,
        ).wait()

    fire_gathers(0, 0)

    def pair(gg, carry):
        for b in range(2):
            g = gg * 2 + b
            nb = 1 - b

            @pl.when(g < _NCHUNK - 1)
            def _fill_next():
                @pl.when(g >= 1)
                def _free_buf():
                    wait_store(g - 1, nb)

                fire_gathers(g + 1, nb)

            drain_gathers(g, b)
            fire_store(g, b)
        return carry

    lax.fori_loop(0, _NCHUNK // 2, pair, 0)
    wait_store(_NCHUNK - 2, 0)
    wait_store(_NCHUNK - 1, 1)


@jax.jit
def _embedding_lookup(input_, weight):
    wp = jnp.pad(weight, ((0, 0), (0, _D)))
    idx2 = input_.astype(jnp.int32).reshape(_B // _IW, _IW)
    mesh = plsc.VectorSubcoreMesh(core_axis_name="c", subcore_axis_name="s")
    outp = pl.kernel(
        _body,
        out_type=jax.ShapeDtypeStruct((_B, _W), jnp.float32),
        mesh=mesh,
        scratch_types=[
            pltpu.VMEM((_KALL, _IW), jnp.int32),
            pltpu.VMEM((2 * _C, _W), jnp.float32),
            pltpu.SemaphoreType.DMA,
            pltpu.SemaphoreType.DMA,
        ],
        compiler_params=pltpu.CompilerParams(
            use_tc_tiling_on_sc=True, needs_layout_passes=False
        ),
    )(wp, idx2)
    return outp[:, :_D].reshape(_BATCH, _HIST, _D)


def kernel(input_, weight):
    return _embedding_lookup(input_, weight)
